# baseline (speedup 1.0000x reference)
import functools

import jax
import jax.numpy as jnp
from jax import lax
from jax.experimental import pallas as pl
from jax.experimental.pallas import tpu as pltpu
from jax.experimental.pallas import tpu_sc as plsc


@functools.lru_cache(maxsize=None)
def _make_emb_lookup(S, BATCH, D, BB):
    info = plsc.get_sparse_core_info()
    nc, ns = info.num_cores, info.num_subcores
    nw = nc * ns
    nb_blocks = BATCH // BB
    n_tasks = S * nb_blocks
    tasks_pw = n_tasks // nw
    assert tasks_pw * nw == n_tasks and tasks_pw >= 3 and tasks_pw % 2 == 0
    nbt = BB // 128
    rpf = nbt * 8 + 1
    n_rows = (D // 8) * rpf
    FI_STR = 129
    Dh = D // 2
    mesh = plsc.VectorSubcoreMesh(core_axis_name="c", subcore_axis_name="s")

    @functools.partial(
        pl.kernel,
        mesh=mesh,
        compiler_params=pltpu.CompilerParams(
            use_tc_tiling_on_sc=False, needs_layout_passes=False
        ),
        out_type=jax.ShapeDtypeStruct(
            (S, (D // 8) * (BATCH // 128) * 8, 128), jnp.float32
        ),
        scratch_types=[
            pltpu.VMEM((2, BB), jnp.int32),
            pltpu.VMEM((2, BB, Dh), jnp.float32),
            pltpu.VMEM((2, BB, Dh), jnp.float32),
            pltpu.VMEM((2, n_rows, FI_STR), jnp.float32),
            pltpu.SemaphoreType.DMA,
            pltpu.SemaphoreType.DMA,
            pltpu.SemaphoreType.DMA,
            pltpu.SemaphoreType.DMA,
        ],
    )
    def emb(xt_hbm, ta_hbm, tb_hbm, out_hbm,
            i2_v, ga_v, gb_v, o_v, sg0, sg1, so0, so1):
        wid = lax.axis_index("s") * nc + lax.axis_index("c")
        t0 = wid * tasks_pw
        sg = (sg0, sg1)
        so = (so0, so1)
        lane = lax.iota(jnp.int32, 16)
        rb_base = (lane >> 3) * rpf + (lane & 7)

        def task_pos(k):
            s = (t0 + k) // nb_blocks
            bb = (t0 + k) % nb_blocks
            return s, bb

        def prep(k, p):
            s, bb = task_pos(k)
            pltpu.sync_copy(xt_hbm.at[s, pl.ds(bb * BB, BB)], i2_v.at[p])

        def gathers(p):
            return (
                pltpu.make_async_copy(ta_hbm.at[i2_v.at[p]], ga_v.at[p], sg[p]),
                pltpu.make_async_copy(tb_hbm.at[i2_v.at[p]], gb_v.at[p], sg[p]),
            )

        def transpose(p):
            o2 = o_v.at[p]
            halves = (ga_v, gb_v)

            for bt in range(nbt):
                rows_bt = tuple(
                    rb_base + (bt * 8 + fb * 2 * rpf) for fb in range(D // 16)
                )

                def cbody(ci, _):
                    col = jnp.broadcast_to(ci, (16,))
                    j = bt * 128 + ci
                    for fb in range(D // 16):
                        g = halves[fb // (Dh // 16)]
                        fo = (fb % (Dh // 16)) * 16
                        vals = g[p, j, pl.ds(fo, 16)]
                        plsc.store_scatter(o2, [rows_bt[fb], col], vals)
                    return 0

                lax.fori_loop(0, 128, cbody, 0)

        def out_copies(k, p):
            s, bb = task_pos(k)
            return [
                pltpu.make_async_copy(
                    o_v.at[p, pl.ds(ft * rpf, nbt * 8), pl.ds(0, 128)],
                    out_hbm.at[
                        s,
                        pl.ds(ft * (BATCH // 128) * 8 + bb * nbt * 8, nbt * 8),
                        :,
                    ],
                    so[p],
                )
                for ft in range(D // 8)
            ]

        prep(0, 0)
        for g in gathers(0):
            g.start()

        def step(k, p, q):
            for g in gathers(p):
                g.wait()

            @pl.when(k + 1 < tasks_pw)
            def _():
                prep(k + 1, q)
                for g in gathers(q):
                    g.start()

            @pl.when(k >= 2)
            def _():
                for c in out_copies(k - 2, p):
                    c.wait()

            transpose(p)
            for c in out_copies(k, p):
                c.start()

        def body(k2, carry):
            step(2 * k2, 0, 1)
            step(2 * k2 + 1, 1, 0)
            return carry

        lax.fori_loop(0, tasks_pw // 2, body, 0)

        for c in out_copies(tasks_pw - 2, (tasks_pw - 2) % 2):
            c.wait()
        for c in out_copies(tasks_pw - 1, (tasks_pw - 1) % 2):
            c.wait()

    return emb


def kernel(x, table):
    bsz, seq = x.shape
    V, D = table.shape
    xt = x.T
    t_a = table[:, : D // 2]
    t_b = table[:, D // 2:]
    out3 = _make_emb_lookup(seq, bsz, D, 256)(xt, t_a, t_b)
    out5 = out3.reshape(seq, D // 8, bsz // 128, 8, 128)
    return out5.transpose(2, 4, 0, 1, 3).reshape(bsz, seq, D)


# speedup vs baseline: 1.7509x; 1.7509x over previous
import functools

import jax
import jax.numpy as jnp
from jax import lax
from jax.experimental import pallas as pl
from jax.experimental.pallas import tpu as pltpu
from jax.experimental.pallas import tpu_sc as plsc


@functools.lru_cache(maxsize=None)
def _make_emb_lookup(S, BATCH, D, BB):
    info = plsc.get_sparse_core_info()
    nc, ns = info.num_cores, info.num_subcores
    nw = nc * ns
    nb_blocks = BATCH // BB
    n_tasks = S * nb_blocks
    tasks_pw = n_tasks // nw
    assert tasks_pw * nw == n_tasks and tasks_pw >= 3 and tasks_pw % 2 == 0
    nbt = BB // 128
    rpf = nbt * 8 + 1
    n_rows = (D // 8) * rpf
    FI_STR = 129
    mesh = plsc.VectorSubcoreMesh(core_axis_name="c", subcore_axis_name="s")

    @functools.partial(
        pl.kernel,
        mesh=mesh,
        compiler_params=pltpu.CompilerParams(
            use_tc_tiling_on_sc=False, needs_layout_passes=False
        ),
        out_type=jax.ShapeDtypeStruct(
            (S, (D // 8) * (BATCH // 128) * 8, 128), jnp.float32
        ),
        scratch_types=[
            pltpu.VMEM((2, BB), jnp.int32),
            pltpu.VMEM((2, BB, D), jnp.float32),
            pltpu.VMEM((2, n_rows, FI_STR), jnp.float32),
            pltpu.SemaphoreType.DMA,
            pltpu.SemaphoreType.DMA,
            pltpu.SemaphoreType.DMA,
            pltpu.SemaphoreType.DMA,
        ],
    )
    def emb(xt_hbm, tab_hbm, out_hbm, i2_v, g_v, o_v, sg0, sg1, so0, so1):
        wid = lax.axis_index("s") * nc + lax.axis_index("c")
        t0 = wid * tasks_pw
        sg = (sg0, sg1)
        so = (so0, so1)
        lane = lax.iota(jnp.int32, 16)
        rb_base = (lane >> 3) * rpf + (lane & 7)

        def task_pos(k):
            s = (t0 + k) // nb_blocks
            bb = (t0 + k) % nb_blocks
            return s, bb

        def prep(k, p):
            s, bb = task_pos(k)
            pltpu.sync_copy(xt_hbm.at[s, pl.ds(bb * BB, BB)], i2_v.at[p])

        def gather(p):
            return pltpu.make_async_copy(tab_hbm.at[i2_v.at[p]], g_v.at[p], sg[p])

        def transpose(p):
            o2 = o_v.at[p]

            def jbody(j4, _):
                for jj in range(4):
                    j = j4 * 4 + jj
                    col = jnp.broadcast_to(j & 127, (16,))
                    roff = (j >> 7) * 8
                    for fb in range(D // 16):
                        vals = g_v[p, j, pl.ds(fb * 16, 16)]
                        rows = rb_base + (roff + fb * 2 * rpf)
                        plsc.store_scatter(o2, [rows, col], vals)
                return 0

            lax.fori_loop(0, BB // 4, jbody, 0)

        def out_copies(k, p):
            s, bb = task_pos(k)
            return [
                pltpu.make_async_copy(
                    o_v.at[p, pl.ds(ft * rpf, nbt * 8), pl.ds(0, 128)],
                    out_hbm.at[
                        s,
                        pl.ds(ft * (BATCH // 128) * 8 + bb * nbt * 8, nbt * 8),
                        :,
                    ],
                    so[p],
                )
                for ft in range(D // 8)
            ]

        prep(0, 0)
        gather(0).start()

        def step(k, p, q):
            gather(p).wait()

            @pl.when(k + 1 < tasks_pw)
            def _():
                prep(k + 1, q)
                gather(q).start()

            @pl.when(k >= 2)
            def _():
                for c in out_copies(k - 2, p):
                    c.wait()

            transpose(p)
            for c in out_copies(k, p):
                c.start()

        def body(k2, carry):
            step(2 * k2, 0, 1)
            step(2 * k2 + 1, 1, 0)
            return carry

        lax.fori_loop(0, tasks_pw // 2, body, 0)

        for c in out_copies(tasks_pw - 2, (tasks_pw - 2) % 2):
            c.wait()
        for c in out_copies(tasks_pw - 1, (tasks_pw - 1) % 2):
            c.wait()

    return emb


def kernel(x, table):
    bsz, seq = x.shape
    V, D = table.shape
    xt = x.T
    out3 = _make_emb_lookup(seq, bsz, D, 256)(xt, table)
    out5 = out3.reshape(seq, D // 8, bsz // 128, 8, 128)
    return out5.transpose(2, 4, 0, 1, 3).reshape(bsz, seq, D)


# speedup vs baseline: 1.8745x; 1.0706x over previous
import functools

import jax
import jax.numpy as jnp
from jax import lax
from jax.experimental import pallas as pl
from jax.experimental.pallas import tpu as pltpu
from jax.experimental.pallas import tpu_sc as plsc


@functools.lru_cache(maxsize=None)
def _make_emb_lookup(S, BATCH, D, BB):
    info = plsc.get_sparse_core_info()
    nc, ns = info.num_cores, info.num_subcores
    nw = nc * ns
    nb_blocks = BATCH // BB
    n_tasks = S * nb_blocks
    tasks_pw = n_tasks // nw
    assert tasks_pw * nw == n_tasks and tasks_pw >= 3 and tasks_pw % 2 == 0
    nbt = BB // 128
    rpf = nbt * 8 + 1
    n_rows = (D // 8) * rpf
    FI_STR = 129
    mesh = plsc.VectorSubcoreMesh(core_axis_name="c", subcore_axis_name="s")

    @functools.partial(
        pl.kernel,
        mesh=mesh,
        compiler_params=pltpu.CompilerParams(
            use_tc_tiling_on_sc=False, needs_layout_passes=False
        ),
        out_type=jax.ShapeDtypeStruct(
            (S, (D // 8) * (BATCH // 128) * 8, 128), jnp.float32
        ),
        scratch_types=[
            pltpu.VMEM((2, BB), jnp.int32),
            pltpu.VMEM((2, BB, D), jnp.float32),
            pltpu.VMEM((2, n_rows, FI_STR), jnp.float32),
            pltpu.SemaphoreType.DMA,
            pltpu.SemaphoreType.DMA,
            pltpu.SemaphoreType.DMA,
            pltpu.SemaphoreType.DMA,
            pltpu.SemaphoreType.DMA,
            pltpu.SemaphoreType.DMA,
        ],
    )
    def emb(xt_hbm, tab_hbm, out_hbm, i2_v, g_v, o_v,
            sg0, sg1, so0, so1, si0, si1):
        wid = lax.axis_index("s") * nc + lax.axis_index("c")
        t0 = wid * tasks_pw
        sg = (sg0, sg1)
        so = (so0, so1)
        si = (si0, si1)
        lane = lax.iota(jnp.int32, 16)
        rb_base = (lane >> 3) * rpf + (lane & 7)

        def task_pos(k):
            s = (t0 + k) // nb_blocks
            bb = (t0 + k) % nb_blocks
            return s, bb

        def prep(k, p):
            s, bb = task_pos(k)
            return pltpu.make_async_copy(
                xt_hbm.at[s, pl.ds(bb * BB, BB)], i2_v.at[p], si[p]
            )

        def gather(p):
            return pltpu.make_async_copy(tab_hbm.at[i2_v.at[p]], g_v.at[p], sg[p])

        def transpose(p):
            o2 = o_v.at[p]

            def jbody(j4, _):
                for jj in range(4):
                    j = j4 * 4 + jj
                    col = jnp.broadcast_to(j & 127, (16,))
                    roff = (j >> 7) * 8
                    for fb in range(D // 16):
                        vals = g_v[p, j, pl.ds(fb * 16, 16)]
                        rows = rb_base + (roff + fb * 2 * rpf)
                        plsc.store_scatter(o2, [rows, col], vals)
                return 0

            lax.fori_loop(0, BB // 4, jbody, 0)

        def out_copies(k, p):
            s, bb = task_pos(k)
            return [
                pltpu.make_async_copy(
                    o_v.at[p, pl.ds(ft * rpf, nbt * 8), pl.ds(0, 128)],
                    out_hbm.at[
                        s,
                        pl.ds(ft * (BATCH // 128) * 8 + bb * nbt * 8, nbt * 8),
                        :,
                    ],
                    so[p],
                )
                for ft in range(D // 8)
            ]

        p0 = prep(0, 0)
        p0.start()
        p0.wait()
        gather(0).start()
        prep(1, 1).start()

        def step(k, p, q):
            gather(p).wait()

            @pl.when(k + 1 < tasks_pw)
            def _():
                prep(k + 1, q).wait()
                gather(q).start()

            @pl.when(k + 2 < tasks_pw)
            def _():
                prep(k + 2, p).start()

            @pl.when(k >= 2)
            def _():
                for c in out_copies(k - 2, p):
                    c.wait()

            transpose(p)
            for c in out_copies(k, p):
                c.start()

        def body(k2, carry):
            step(2 * k2, 0, 1)
            step(2 * k2 + 1, 1, 0)
            return carry

        lax.fori_loop(0, tasks_pw // 2, body, 0)

        for c in out_copies(tasks_pw - 2, (tasks_pw - 2) % 2):
            c.wait()
        for c in out_copies(tasks_pw - 1, (tasks_pw - 1) % 2):
            c.wait()

    return emb


def kernel(x, table):
    bsz, seq = x.shape
    V, D = table.shape
    xt = x.T
    out3 = _make_emb_lookup(seq, bsz, D, 256)(xt, table)
    out5 = out3.reshape(seq, D // 8, bsz // 128, 8, 128)
    return out5.transpose(2, 4, 0, 1, 3).reshape(bsz, seq, D)
